# baseline (speedup 1.0000x reference)
import functools

import jax
import jax.numpy as jnp
from jax import lax
from jax.experimental import pallas as pl
from jax.experimental.pallas import tpu as pltpu
from jax.experimental.pallas import tpu_sc as plsc

N_NODES = 10000
N_EDGES = 320000
D_IN = 128
D_OUT = 128

NC = 2
NS = 16
NW = NC * NS
CH = 64
DEXT = D_IN + 16
NACC = 10112
ROWS_PER_TILE = NACC // NS
CPT = 8 * (-(-N_EDGES // (NW * CH * 8)))
EPT = CPT * CH
EPAD = EPT * NW
NRB = 4
NIB = 8


def _sc_body(hext_hbm, idx_hbm, zeros_hbm, acc_out,
             rows, idxb, acc_sh, gsems, ssems, isems):
    c = lax.axis_index("c")
    s = lax.axis_index("s")
    wid = s * NC + c
    my_rows = pl.ds(s * ROWS_PER_TILE, ROWS_PER_TILE)
    pltpu.sync_copy(zeros_hbm, acc_sh.at[my_rows])
    plsc.subcore_barrier()

    def steps(j, dj):
        k4 = dj % NRB
        k8 = dj % NIB

        @pl.when((j >= 4) & (j < CPT + 4))
        def _drain_scatter():
            kk8 = (dj - 4) % NIB
            pltpu.make_async_copy(
                rows[k4], acc_sh.at[idxb[kk8].at[1]], ssems[k4]).wait()

        @pl.when(j < CPT)
        def _gather():
            pltpu.make_async_copy(idx_hbm.at[wid, j], idxb[k8],
                                  isems[k8]).wait()
            pltpu.async_copy(hext_hbm.at[idxb[k8].at[0]], rows[k4], gsems[k4])

        @pl.when(j + 4 < CPT)
        def _prefetch_idx():
            kk8 = (dj + 4) % NIB
            pltpu.async_copy(idx_hbm.at[wid, j + 4], idxb[kk8], isems[kk8])

        @pl.when((j >= 2) & (j < CPT + 2))
        def _scatter():
            kk4 = (dj - 2) % NRB
            kk8 = (dj - 2) % NIB
            pltpu.make_async_copy(
                hext_hbm.at[idxb[kk8].at[0]], rows[kk4], gsems[kk4]).wait()
            pltpu.async_copy(rows[kk4], acc_sh.at[idxb[kk8].at[1]],
                             ssems[kk4], add=True)

    for k in range(4):
        pltpu.async_copy(idx_hbm.at[wid, k], idxb[k], isems[k])

    def block(bi, carry):
        j0 = bi * 8
        for dj in range(8):
            steps(j0 + dj, dj)
        return carry

    lax.fori_loop(0, (CPT + 8) // 8, block, 0)
    plsc.subcore_barrier()
    pltpu.sync_copy(acc_sh.at[my_rows], acc_out.at[c, my_rows])


def _sc_aggregate(hext, idx4, zeros_slab):
    mesh = plsc.VectorSubcoreMesh(core_axis_name="c", subcore_axis_name="s")
    f = pl.kernel(
        _sc_body,
        out_type=jax.ShapeDtypeStruct((NC, NACC, DEXT), jnp.float32),
        mesh=mesh,
        compiler_params=pltpu.CompilerParams(use_tc_tiling_on_sc=False),
        scratch_types=[
            [pltpu.VMEM((CH, DEXT), jnp.float32) for _ in range(NRB)],
            [pltpu.VMEM((2, CH), jnp.int32) for _ in range(NIB)],
            pltpu.VMEM_SHARED((NACC, DEXT), jnp.float32),
            [pltpu.SemaphoreType.DMA for _ in range(NRB)],
            [pltpu.SemaphoreType.DMA for _ in range(NRB)],
            [pltpu.SemaphoreType.DMA for _ in range(NIB)],
        ],
    )
    return f(hext, idx4, zeros_slab)


def _tc_body(h_ref, acc_ref, w_ref, b_ref, o_ref):
    a0 = acc_ref[0]
    a1 = acc_ref[1]
    sums = a0[:, :D_IN] + a1[:, :D_IN]
    cnt = a0[:, D_IN:D_IN + 1] + a1[:, D_IN:D_IN + 1]
    neigh = sums / jnp.maximum(cnt, 1.0)
    r = lax.dot_general(h_ref[...], w_ref[:, :D_IN],
                        (((1,), (1,)), ((), ())),
                        preferred_element_type=jnp.float32)
    r = r + lax.dot_general(neigh, w_ref[:, D_IN:],
                            (((1,), (1,)), ((), ())),
                            preferred_element_type=jnp.float32)
    o_ref[...] = jnp.maximum(r + b_ref[...], 0.0)


def _tc_finish(h_pad, acc, W, b2, interpret=False):
    R = 632
    grid = (NACC // R,)
    return pl.pallas_call(
        _tc_body,
        grid=grid,
        in_specs=[
            pl.BlockSpec((R, D_IN), lambda i: (i, 0)),
            pl.BlockSpec((NC, R, DEXT), lambda i: (0, i, 0)),
            pl.BlockSpec((D_IN, 2 * D_IN), lambda i: (0, 0)),
            pl.BlockSpec((1, D_OUT), lambda i: (0, 0)),
        ],
        out_specs=pl.BlockSpec((R, D_OUT), lambda i: (i, 0)),
        out_shape=jax.ShapeDtypeStruct((NACC, D_OUT), jnp.float32),
        interpret=interpret,
    )(h_pad, acc, W, b2)


def kernel(h, edge_index, W, b):
    src = edge_index[0].astype(jnp.int32)
    dst = edge_index[1].astype(jnp.int32)
    pad = EPAD - N_EDGES
    src3 = jnp.concatenate([src, jnp.zeros((pad,), jnp.int32)]).reshape(NW, CPT, 1, CH)
    dst_pad = N_NODES + (jnp.arange(pad, dtype=jnp.int32) % (NACC - N_NODES))
    dst3 = jnp.concatenate([dst, dst_pad]).reshape(NW, CPT, 1, CH)
    idx4 = jnp.concatenate([src3, dst3], axis=2)
    ones_col = jnp.concatenate(
        [jnp.ones((N_NODES, 1), jnp.float32),
         jnp.zeros((N_NODES, DEXT - D_IN - 1), jnp.float32)], axis=1)
    hext = jnp.concatenate([h, ones_col], axis=1)
    zeros_slab = jnp.zeros((ROWS_PER_TILE, DEXT), jnp.float32)

    acc = _sc_aggregate(hext, idx4, zeros_slab)

    h_pad = jnp.concatenate([h, jnp.zeros((NACC - N_NODES, D_IN), jnp.float32)])
    out = _tc_finish(h_pad, acc, W, b.reshape(1, D_OUT))
    return out[:N_NODES]


# speedup vs baseline: 1.3981x; 1.3981x over previous
import functools

import jax
import jax.numpy as jnp
from jax import lax
from jax.experimental import pallas as pl
from jax.experimental.pallas import tpu as pltpu
from jax.experimental.pallas import tpu_sc as plsc

N_NODES = 10000
N_EDGES = 320000
D_IN = 128
D_OUT = 128

NC = 2
NS = 16
NW = NC * NS
CH = 128
DEXT = D_IN + 16
NACC = 10112
ROWS_PER_TILE = NACC // NS
CPT = 2 * (-(-N_EDGES // (NW * CH * 2)))
EPT = CPT * CH
EPAD = EPT * NW


def _sc_body(hext_hbm, src_hbm, dst_hbm, zeros_hbm, acc_out,
             src_v, dst_v, rows_a, rows_b, acc_sh, gsems, ssems):
    c = lax.axis_index("c")
    s = lax.axis_index("s")
    wid = s * NC + c
    my_rows = pl.ds(s * ROWS_PER_TILE, ROWS_PER_TILE)
    pltpu.sync_copy(zeros_hbm, acc_sh.at[my_rows])
    pltpu.sync_copy(src_hbm.at[wid], src_v)
    pltpu.sync_copy(dst_hbm.at[wid], dst_v)
    plsc.subcore_barrier()

    def fire_gather(j, buf, sem):
        pltpu.async_copy(hext_hbm.at[src_v.at[j]], buf, sem)

    def wait_gather(j, buf, sem):
        pltpu.make_async_copy(hext_hbm.at[src_v.at[j]], buf, sem).wait()

    def fire_scatter(j, buf, sem):
        pltpu.async_copy(buf, acc_sh.at[dst_v.at[j]], sem, add=True)

    def wait_scatter(j, buf, sem):
        pltpu.make_async_copy(buf, acc_sh.at[dst_v.at[j]], sem).wait()

    def step(i, carry):
        j0 = 2 * i
        @pl.when(i >= 1)
        def _():
            wait_scatter(j0 - 1, rows_b, ssems[1])
        fire_gather(j0 + 1, rows_b, gsems[1])
        wait_gather(j0, rows_a, gsems[0])
        fire_scatter(j0, rows_a, ssems[0])

        wait_scatter(j0, rows_a, ssems[0])
        @pl.when(i < CPT // 2 - 1)
        def _():
            fire_gather(j0 + 2, rows_a, gsems[0])
        wait_gather(j0 + 1, rows_b, gsems[1])
        fire_scatter(j0 + 1, rows_b, ssems[1])
        return carry

    fire_gather(0, rows_a, gsems[0])
    lax.fori_loop(0, CPT // 2, step, 0)
    wait_scatter(CPT - 1, rows_b, ssems[1])
    plsc.subcore_barrier()
    pltpu.sync_copy(acc_sh.at[my_rows], acc_out.at[c, my_rows])


def _sc_aggregate(hext, src3, dst3, zeros_slab):
    mesh = plsc.VectorSubcoreMesh(core_axis_name="c", subcore_axis_name="s")
    f = pl.kernel(
        _sc_body,
        out_type=jax.ShapeDtypeStruct((NC, NACC, DEXT), jnp.bfloat16),
        mesh=mesh,
        compiler_params=pltpu.CompilerParams(use_tc_tiling_on_sc=False),
        scratch_types=[
            pltpu.VMEM((CPT, CH), jnp.int32),
            pltpu.VMEM((CPT, CH), jnp.int32),
            pltpu.VMEM((CH, DEXT), jnp.bfloat16),
            pltpu.VMEM((CH, DEXT), jnp.bfloat16),
            pltpu.VMEM_SHARED((NACC, DEXT), jnp.bfloat16),
            [pltpu.SemaphoreType.DMA for _ in range(2)],
            [pltpu.SemaphoreType.DMA for _ in range(2)],
        ],
    )
    return f(hext, src3, dst3, zeros_slab)


def _tc_body(h_ref, acc_ref, w_ref, b_ref, o_ref):
    a0 = acc_ref[0].astype(jnp.float32)
    a1 = acc_ref[1].astype(jnp.float32)
    sums = a0[:, :D_IN] + a1[:, :D_IN]
    cnt = a0[:, D_IN:D_IN + 1] + a1[:, D_IN:D_IN + 1]
    neigh = sums / jnp.maximum(cnt, 1.0)
    r = lax.dot_general(h_ref[...], w_ref[:, :D_IN],
                        (((1,), (1,)), ((), ())),
                        preferred_element_type=jnp.float32)
    r = r + lax.dot_general(neigh, w_ref[:, D_IN:],
                            (((1,), (1,)), ((), ())),
                            preferred_element_type=jnp.float32)
    o_ref[...] = jnp.maximum(r + b_ref[...], 0.0)


def _tc_finish(h_pad, acc, W, b2, interpret=False):
    R = 632
    grid = (NACC // R,)
    return pl.pallas_call(
        _tc_body,
        grid=grid,
        in_specs=[
            pl.BlockSpec((R, D_IN), lambda i: (i, 0)),
            pl.BlockSpec((NC, R, DEXT), lambda i: (0, i, 0)),
            pl.BlockSpec((D_IN, 2 * D_IN), lambda i: (0, 0)),
            pl.BlockSpec((1, D_OUT), lambda i: (0, 0)),
        ],
        out_specs=pl.BlockSpec((R, D_OUT), lambda i: (i, 0)),
        out_shape=jax.ShapeDtypeStruct((NACC, D_OUT), jnp.float32),
        interpret=interpret,
    )(h_pad, acc, W, b2)


def kernel(h, edge_index, W, b):
    src = edge_index[0].astype(jnp.int32)
    dst = edge_index[1].astype(jnp.int32)
    pad = EPAD - N_EDGES
    src3 = jnp.concatenate([src, jnp.zeros((pad,), jnp.int32)]).reshape(NW, CPT, CH)
    dst_pad = N_NODES + (jnp.arange(pad, dtype=jnp.int32) % (NACC - N_NODES))
    dst3 = jnp.concatenate([dst, dst_pad]).reshape(NW, CPT, CH)
    ones_col = jnp.concatenate(
        [jnp.ones((N_NODES, 1), jnp.float32),
         jnp.zeros((N_NODES, DEXT - D_IN - 1), jnp.float32)], axis=1)
    hext = jnp.concatenate([h, ones_col], axis=1).astype(jnp.bfloat16)
    zeros_slab = jnp.zeros((ROWS_PER_TILE, DEXT), jnp.bfloat16)

    acc = _sc_aggregate(hext, src3, dst3, zeros_slab)

    h_pad = jnp.concatenate([h, jnp.zeros((NACC - N_NODES, D_IN), jnp.float32)])
    out = _tc_finish(h_pad, acc, W, b.reshape(1, D_OUT))
    return out[:N_NODES]


# speedup vs baseline: 1.5221x; 1.0887x over previous
import functools

import jax
import jax.numpy as jnp
from jax import lax
from jax.experimental import pallas as pl
from jax.experimental.pallas import tpu as pltpu
from jax.experimental.pallas import tpu_sc as plsc

N_NODES = 10000
N_EDGES = 320000
D_IN = 128
D_OUT = 128

NC = 2
NS = 16
NW = NC * NS
CH = 128
DEXT = D_IN + 16
NACC = 10112
ROWS_PER_TILE = NACC // NS
CPT = 2 * (-(-N_EDGES // (NW * CH * 2)))
EPT = CPT * CH
EPAD = EPT * NW
NRB = 4


def _sc_body(hext_hbm, src_hbm, dst_hbm, zeros_hbm, acc_out,
             src_v, dst_v, rows, acc_sh, gsems, ssems):
    c = lax.axis_index("c")
    s = lax.axis_index("s")
    wid = s * NC + c
    my_rows = pl.ds(s * ROWS_PER_TILE, ROWS_PER_TILE)
    pltpu.sync_copy(zeros_hbm, acc_sh.at[my_rows])
    pltpu.sync_copy(src_hbm.at[wid], src_v)
    pltpu.sync_copy(dst_hbm.at[wid], dst_v)
    plsc.subcore_barrier()

    def steps(j, dj):
        k = dj % NRB

        @pl.when((j >= 4) & (j < CPT + 4))
        def _drain_scatter():
            pltpu.make_async_copy(
                rows[k], acc_sh.at[dst_v.at[j - 4]], ssems[k]).wait()

        @pl.when(j < CPT)
        def _gather():
            pltpu.async_copy(hext_hbm.at[src_v.at[j]], rows[k], gsems[k])

        @pl.when((j >= 2) & (j < CPT + 2))
        def _scatter():
            kk = (dj - 2) % NRB
            pltpu.make_async_copy(
                hext_hbm.at[src_v.at[j - 2]], rows[kk], gsems[kk]).wait()
            pltpu.async_copy(rows[kk], acc_sh.at[dst_v.at[j - 2]],
                             ssems[kk], add=True)

    def block(bi, carry):
        j0 = bi * NRB
        for dj in range(NRB):
            steps(j0 + dj, dj)
        return carry

    lax.fori_loop(0, (CPT + 4 + NRB - 1) // NRB + 1, block, 0)
    plsc.subcore_barrier()
    pltpu.sync_copy(acc_sh.at[my_rows], acc_out.at[c, my_rows])


def _sc_aggregate(hext, src3, dst3, zeros_slab):
    mesh = plsc.VectorSubcoreMesh(core_axis_name="c", subcore_axis_name="s")
    f = pl.kernel(
        _sc_body,
        out_type=jax.ShapeDtypeStruct((NC, NACC, DEXT), jnp.bfloat16),
        mesh=mesh,
        compiler_params=pltpu.CompilerParams(use_tc_tiling_on_sc=False),
        scratch_types=[
            pltpu.VMEM((CPT, CH), jnp.int32),
            pltpu.VMEM((CPT, CH), jnp.int32),
            [pltpu.VMEM((CH, DEXT), jnp.bfloat16) for _ in range(NRB)],
            pltpu.VMEM_SHARED((NACC, DEXT), jnp.bfloat16),
            [pltpu.SemaphoreType.DMA for _ in range(NRB)],
            [pltpu.SemaphoreType.DMA for _ in range(NRB)],
        ],
    )
    return f(hext, src3, dst3, zeros_slab)


def _tc_body(h_ref, acc_ref, w_ref, b_ref, o_ref):
    a0 = acc_ref[0].astype(jnp.float32)
    a1 = acc_ref[1].astype(jnp.float32)
    sums = a0[:, :D_IN] + a1[:, :D_IN]
    cnt = a0[:, D_IN:D_IN + 1] + a1[:, D_IN:D_IN + 1]
    neigh = sums / jnp.maximum(cnt, 1.0)
    r = lax.dot_general(h_ref[...], w_ref[:, :D_IN],
                        (((1,), (1,)), ((), ())),
                        preferred_element_type=jnp.float32)
    r = r + lax.dot_general(neigh, w_ref[:, D_IN:],
                            (((1,), (1,)), ((), ())),
                            preferred_element_type=jnp.float32)
    o_ref[...] = jnp.maximum(r + b_ref[...], 0.0)


def _tc_finish(h_pad, acc, W, b2, interpret=False):
    R = 632
    grid = (NACC // R,)
    return pl.pallas_call(
        _tc_body,
        grid=grid,
        in_specs=[
            pl.BlockSpec((R, D_IN), lambda i: (i, 0)),
            pl.BlockSpec((NC, R, DEXT), lambda i: (0, i, 0)),
            pl.BlockSpec((D_IN, 2 * D_IN), lambda i: (0, 0)),
            pl.BlockSpec((1, D_OUT), lambda i: (0, 0)),
        ],
        out_specs=pl.BlockSpec((R, D_OUT), lambda i: (i, 0)),
        out_shape=jax.ShapeDtypeStruct((NACC, D_OUT), jnp.float32),
        interpret=interpret,
    )(h_pad, acc, W, b2)


def kernel(h, edge_index, W, b):
    src = edge_index[0].astype(jnp.int32)
    dst = edge_index[1].astype(jnp.int32)
    pad = EPAD - N_EDGES
    src3 = jnp.concatenate([src, jnp.zeros((pad,), jnp.int32)]).reshape(NW, CPT, CH)
    dst_pad = N_NODES + (jnp.arange(pad, dtype=jnp.int32) % (NACC - N_NODES))
    dst3 = jnp.concatenate([dst, dst_pad]).reshape(NW, CPT, CH)
    ones_col = jnp.concatenate(
        [jnp.ones((N_NODES, 1), jnp.float32),
         jnp.zeros((N_NODES, DEXT - D_IN - 1), jnp.float32)], axis=1)
    hext = jnp.concatenate([h, ones_col], axis=1).astype(jnp.bfloat16)
    zeros_slab = jnp.zeros((ROWS_PER_TILE, DEXT), jnp.bfloat16)

    acc = _sc_aggregate(hext, src3, dst3, zeros_slab)

    h_pad = jnp.concatenate([h, jnp.zeros((NACC - N_NODES, D_IN), jnp.float32)])
    out = _tc_finish(h_pad, acc, W, b.reshape(1, D_OUT))
    return out[:N_NODES]


# speedup vs baseline: 3.0979x; 2.0352x over previous
import functools

import jax
import jax.numpy as jnp
from jax import lax
from jax.experimental import pallas as pl
from jax.experimental.pallas import tpu as pltpu
from jax.experimental.pallas import tpu_sc as plsc

N_NODES = 10000
N_EDGES = 320000
D_IN = 128
D_OUT = 128

NC = 2
NS = 16
NW = NC * NS
CH = 128
DEXT = D_IN + 16
NACC = 10112
ROWS_PER_TILE = NACC // NS
CPT = 2 * (-(-N_EDGES // (NW * CH * 2)))
EPT = CPT * CH
EPAD = EPT * NW
NRB = 4


def _sc_body(hext_hbm, src_hbm, dst_hbm, zeros_hbm, acc_out,
             src_v, dst_v, rows, acc_sh, gsems, ssems):
    c = lax.axis_index("c")
    s = lax.axis_index("s")
    wid = s * NC + c
    my_rows = pl.ds(s * ROWS_PER_TILE, ROWS_PER_TILE)
    pltpu.sync_copy(zeros_hbm, acc_sh.at[my_rows])
    pltpu.sync_copy(src_hbm.at[wid], src_v)
    pltpu.sync_copy(dst_hbm.at[wid], dst_v)
    plsc.subcore_barrier()

    def steps(j, dj):
        k = dj % NRB

        @pl.when((j >= 4) & (j < CPT + 4))
        def _drain_scatter():
            pltpu.make_async_copy(
                rows[k], acc_sh.at[dst_v.at[j - 4]], ssems[k]).wait()

        @pl.when(j < CPT)
        def _gather():
            pltpu.async_copy(hext_hbm.at[src_v.at[j]], rows[k], gsems[k])

        @pl.when((j >= 2) & (j < CPT + 2))
        def _scatter():
            kk = (dj - 2) % NRB
            pltpu.make_async_copy(
                hext_hbm.at[src_v.at[j - 2]], rows[kk], gsems[kk]).wait()
            pltpu.async_copy(rows[kk], acc_sh.at[dst_v.at[j - 2]],
                             ssems[kk], add=True)

    def block(bi, carry):
        j0 = bi * NRB
        for dj in range(NRB):
            steps(j0 + dj, dj)
        return carry

    lax.fori_loop(0, (CPT + 4 + NRB - 1) // NRB + 1, block, 0)
    plsc.subcore_barrier()
    pltpu.sync_copy(acc_sh.at[my_rows], acc_out.at[c, my_rows])


def _sc_aggregate(hext, src3, dst3, zeros_slab):
    mesh = plsc.VectorSubcoreMesh(core_axis_name="c", subcore_axis_name="s")
    f = pl.kernel(
        _sc_body,
        out_type=jax.ShapeDtypeStruct((NC, NACC, DEXT), jnp.bfloat16),
        mesh=mesh,
        compiler_params=pltpu.CompilerParams(use_tc_tiling_on_sc=False),
        scratch_types=[
            pltpu.VMEM((CPT, CH), jnp.int32),
            pltpu.VMEM((CPT, CH), jnp.int32),
            [pltpu.VMEM((CH, DEXT), jnp.bfloat16) for _ in range(NRB)],
            pltpu.VMEM_SHARED((NACC, DEXT), jnp.bfloat16),
            [pltpu.SemaphoreType.DMA for _ in range(NRB)],
            [pltpu.SemaphoreType.DMA for _ in range(NRB)],
        ],
    )
    return f(hext, src3, dst3, zeros_slab)


def _tc_body(h_ref, acc_ref, w_ref, b_ref, o_ref):
    a0 = acc_ref[0].astype(jnp.float32)
    a1 = acc_ref[1].astype(jnp.float32)
    sums = a0[:, :D_IN] + a1[:, :D_IN]
    cnt = a0[:, D_IN:D_IN + 1] + a1[:, D_IN:D_IN + 1]
    neigh = sums / jnp.maximum(cnt, 1.0)
    r = lax.dot_general(h_ref[...], w_ref[:, :D_IN],
                        (((1,), (1,)), ((), ())),
                        preferred_element_type=jnp.float32)
    r = r + lax.dot_general(neigh, w_ref[:, D_IN:],
                            (((1,), (1,)), ((), ())),
                            preferred_element_type=jnp.float32)
    o_ref[...] = jnp.maximum(r + b_ref[...], 0.0)


def _tc_finish(h_pad, acc, W, b2, interpret=False):
    R = 632
    grid = (NACC // R,)
    return pl.pallas_call(
        _tc_body,
        grid=grid,
        in_specs=[
            pl.BlockSpec((R, D_IN), lambda i: (i, 0)),
            pl.BlockSpec((NC, R, DEXT), lambda i: (0, i, 0)),
            pl.BlockSpec((D_IN, 2 * D_IN), lambda i: (0, 0)),
            pl.BlockSpec((1, D_OUT), lambda i: (0, 0)),
        ],
        out_specs=pl.BlockSpec((R, D_OUT), lambda i: (i, 0)),
        out_shape=jax.ShapeDtypeStruct((NACC, D_OUT), jnp.float32),
        interpret=interpret,
    )(h_pad, acc, W, b2)


def kernel(h, edge_index, W, b):
    src = edge_index[0].astype(jnp.int32)
    dst = edge_index[1].astype(jnp.int32)
    ppt = EPT - N_EDGES // NW
    src_pad = (jnp.arange(NW * ppt, dtype=jnp.int32) % N_NODES).reshape(NW, ppt)
    drows = (NACC - N_NODES) // NS
    tile_s = jnp.arange(NW, dtype=jnp.int32) // NC
    dst_pad = (N_NODES + tile_s[:, None] * drows
               + (jnp.arange(ppt, dtype=jnp.int32)[None, :] % drows))
    src3 = jnp.concatenate([src.reshape(NW, -1), src_pad], axis=1
                           ).reshape(NW, CPT, CH)
    dst3 = jnp.concatenate([dst.reshape(NW, -1), dst_pad], axis=1
                           ).reshape(NW, CPT, CH)
    ones_col = jnp.concatenate(
        [jnp.ones((N_NODES, 1), jnp.float32),
         jnp.zeros((N_NODES, DEXT - D_IN - 1), jnp.float32)], axis=1)
    hext = jnp.concatenate([h, ones_col], axis=1).astype(jnp.bfloat16)
    zeros_slab = jnp.zeros((ROWS_PER_TILE, DEXT), jnp.bfloat16)

    acc = _sc_aggregate(hext, src3, dst3, zeros_slab)

    h_pad = jnp.concatenate([h, jnp.zeros((NACC - N_NODES, D_IN), jnp.float32)])
    out = _tc_finish(h_pad, acc, W, b.reshape(1, D_OUT))
    return out[:N_NODES]


# speedup vs baseline: 3.3278x; 1.0742x over previous
import functools

import jax
import jax.numpy as jnp
from jax import lax
from jax.experimental import pallas as pl
from jax.experimental.pallas import tpu as pltpu
from jax.experimental.pallas import tpu_sc as plsc

N_NODES = 10000
N_EDGES = 320000
D_IN = 128
D_OUT = 128

NC = 2
NS = 16
NW = NC * NS
CH = 125
DEXT = D_IN + 16
NACC = N_NODES
ROWS_PER_TILE = NACC // NS
EPT = N_EDGES // NW
CPT = EPT // CH
NRB = 4


def _sc_body(hext_hbm, src_hbm, dst_hbm, zeros_hbm, acc_out,
             src_v, dst_v, rows, acc_sh, gsems, ssems):
    c = lax.axis_index("c")
    s = lax.axis_index("s")
    wid = s * NC + c
    my_rows = pl.ds(s * ROWS_PER_TILE, ROWS_PER_TILE)
    pltpu.sync_copy(zeros_hbm, acc_sh.at[my_rows])
    pltpu.sync_copy(src_hbm.at[wid], src_v)
    pltpu.sync_copy(dst_hbm.at[wid], dst_v)
    plsc.subcore_barrier()

    def steps(j, dj):
        k = dj % NRB

        @pl.when((j >= 4) & (j < CPT + 4))
        def _drain_scatter():
            pltpu.make_async_copy(
                rows[k], acc_sh.at[dst_v.at[j - 4]], ssems[k]).wait()

        @pl.when(j < CPT)
        def _gather():
            pltpu.async_copy(hext_hbm.at[src_v.at[j]], rows[k], gsems[k])

        @pl.when((j >= 2) & (j < CPT + 2))
        def _scatter():
            kk = (dj - 2) % NRB
            pltpu.make_async_copy(
                hext_hbm.at[src_v.at[j - 2]], rows[kk], gsems[kk]).wait()
            pltpu.async_copy(rows[kk], acc_sh.at[dst_v.at[j - 2]],
                             ssems[kk], add=True)

    def block(bi, carry):
        j0 = bi * NRB
        for dj in range(NRB):
            steps(j0 + dj, dj)
        return carry

    lax.fori_loop(0, (CPT + 4 + NRB - 1) // NRB + 1, block, 0)
    plsc.subcore_barrier()
    pltpu.sync_copy(acc_sh.at[my_rows], acc_out.at[c, my_rows])


def _sc_aggregate(hext, src3, dst3, zeros_slab):
    mesh = plsc.VectorSubcoreMesh(core_axis_name="c", subcore_axis_name="s")
    f = pl.kernel(
        _sc_body,
        out_type=jax.ShapeDtypeStruct((NC, NACC, DEXT), jnp.bfloat16),
        mesh=mesh,
        compiler_params=pltpu.CompilerParams(use_tc_tiling_on_sc=False),
        scratch_types=[
            pltpu.VMEM((CPT, CH), jnp.int32),
            pltpu.VMEM((CPT, CH), jnp.int32),
            [pltpu.VMEM((CH, DEXT), jnp.bfloat16) for _ in range(NRB)],
            pltpu.VMEM_SHARED((NACC, DEXT), jnp.bfloat16),
            [pltpu.SemaphoreType.DMA for _ in range(NRB)],
            [pltpu.SemaphoreType.DMA for _ in range(NRB)],
        ],
    )
    return f(hext, src3, dst3, zeros_slab)


def _tc_body(h_ref, acc_ref, w_ref, b_ref, o_ref):
    a0 = acc_ref[0].astype(jnp.float32)
    a1 = acc_ref[1].astype(jnp.float32)
    sums = a0[:, :D_IN] + a1[:, :D_IN]
    cnt = a0[:, D_IN:D_IN + 1] + a1[:, D_IN:D_IN + 1]
    neigh = sums / jnp.maximum(cnt, 1.0)
    r = lax.dot_general(h_ref[...], w_ref[:, :D_IN],
                        (((1,), (1,)), ((), ())),
                        preferred_element_type=jnp.float32)
    r = r + lax.dot_general(neigh, w_ref[:, D_IN:],
                            (((1,), (1,)), ((), ())),
                            preferred_element_type=jnp.float32)
    o_ref[...] = jnp.maximum(r + b_ref[...], 0.0)


def _tc_finish(h_pad, acc, W, b2, interpret=False):
    R = 1000
    grid = (NACC // R,)
    return pl.pallas_call(
        _tc_body,
        grid=grid,
        in_specs=[
            pl.BlockSpec((R, D_IN), lambda i: (i, 0)),
            pl.BlockSpec((NC, R, DEXT), lambda i: (0, i, 0)),
            pl.BlockSpec((D_IN, 2 * D_IN), lambda i: (0, 0)),
            pl.BlockSpec((1, D_OUT), lambda i: (0, 0)),
        ],
        out_specs=pl.BlockSpec((R, D_OUT), lambda i: (i, 0)),
        out_shape=jax.ShapeDtypeStruct((NACC, D_OUT), jnp.float32),
        interpret=interpret,
    )(h_pad, acc, W, b2)


def _hext_body(h_ref, o_ref):
    blk = h_ref[...].astype(jnp.bfloat16)
    ones = jnp.concatenate(
        [jnp.ones(blk.shape[:1] + (1,), jnp.bfloat16),
         jnp.zeros(blk.shape[:1] + (DEXT - D_IN - 1,), jnp.bfloat16)], axis=1)
    o_ref[...] = jnp.concatenate([blk, ones], axis=1)


def _hext_build(h):
    R = 1000
    return pl.pallas_call(
        _hext_body,
        grid=(N_NODES // R,),
        in_specs=[pl.BlockSpec((R, D_IN), lambda i: (i, 0))],
        out_specs=pl.BlockSpec((R, DEXT), lambda i: (i, 0)),
        out_shape=jax.ShapeDtypeStruct((N_NODES, DEXT), jnp.bfloat16),
    )(h)


def kernel(h, edge_index, W, b):
    src3 = edge_index[0].astype(jnp.int32).reshape(NW, CPT, CH)
    dst3 = edge_index[1].astype(jnp.int32).reshape(NW, CPT, CH)
    hext = _hext_build(h)
    zeros_slab = jnp.zeros((ROWS_PER_TILE, DEXT), jnp.bfloat16)

    acc = _sc_aggregate(hext, src3, dst3, zeros_slab)
    return _tc_finish(h, acc, W, b.reshape(1, D_OUT))


# speedup vs baseline: 3.5511x; 1.0671x over previous
import functools

import jax
import jax.numpy as jnp
from jax import lax
from jax.experimental import pallas as pl
from jax.experimental.pallas import tpu as pltpu
from jax.experimental.pallas import tpu_sc as plsc

N_NODES = 10000
N_EDGES = 320000
D_IN = 128
D_OUT = 128

NC = 2
NS = 16
NW = NC * NS
CH = 125
DEXT = D_IN + 16
NACC = N_NODES
ROWS_PER_TILE = NACC // NS
EPT = N_EDGES // NW
CPT = EPT // CH
NRB = 4


def _sc_body(hext_hbm, idx_hbm, zeros_hbm, acc_out,
             src_v, dst_v, rows, acc_sh, gsems, ssems):
    c = lax.axis_index("c")
    s = lax.axis_index("s")
    wid = s * NC + c
    my_rows = pl.ds(s * ROWS_PER_TILE, ROWS_PER_TILE)
    pltpu.sync_copy(zeros_hbm, acc_sh.at[my_rows])
    pltpu.sync_copy(idx_hbm.at[0, wid], src_v)
    pltpu.sync_copy(idx_hbm.at[1, wid], dst_v)
    plsc.subcore_barrier()

    def steps(j, dj):
        k = dj % NRB

        @pl.when((j >= 4) & (j < CPT + 4))
        def _drain_scatter():
            pltpu.make_async_copy(
                rows[k], acc_sh.at[dst_v.at[j - 4]], ssems[k]).wait()

        @pl.when(j < CPT)
        def _gather():
            pltpu.async_copy(hext_hbm.at[src_v.at[j]], rows[k], gsems[k])

        @pl.when((j >= 2) & (j < CPT + 2))
        def _scatter():
            kk = (dj - 2) % NRB
            pltpu.make_async_copy(
                hext_hbm.at[src_v.at[j - 2]], rows[kk], gsems[kk]).wait()
            pltpu.async_copy(rows[kk], acc_sh.at[dst_v.at[j - 2]],
                             ssems[kk], add=True)

    def block(bi, carry):
        j0 = bi * NRB
        for dj in range(NRB):
            steps(j0 + dj, dj)
        return carry

    lax.fori_loop(0, (CPT + 4 + NRB - 1) // NRB + 1, block, 0)
    plsc.subcore_barrier()
    pltpu.sync_copy(acc_sh.at[my_rows], acc_out.at[c, my_rows])


def _sc_aggregate(hext, idx4, zeros_slab):
    mesh = plsc.VectorSubcoreMesh(core_axis_name="c", subcore_axis_name="s")
    f = pl.kernel(
        _sc_body,
        out_type=jax.ShapeDtypeStruct((NC, NACC, DEXT), jnp.bfloat16),
        mesh=mesh,
        compiler_params=pltpu.CompilerParams(use_tc_tiling_on_sc=False),
        scratch_types=[
            pltpu.VMEM((CPT, CH), jnp.int32),
            pltpu.VMEM((CPT, CH), jnp.int32),
            [pltpu.VMEM((CH, DEXT), jnp.bfloat16) for _ in range(NRB)],
            pltpu.VMEM_SHARED((NACC, DEXT), jnp.bfloat16),
            [pltpu.SemaphoreType.DMA for _ in range(NRB)],
            [pltpu.SemaphoreType.DMA for _ in range(NRB)],
        ],
    )
    return f(hext, idx4, zeros_slab)


def _tc_body(h_ref, acc_ref, w_ref, b_ref, o_ref):
    a0 = acc_ref[0].astype(jnp.float32)
    a1 = acc_ref[1].astype(jnp.float32)
    sums = a0[:, :D_IN] + a1[:, :D_IN]
    cnt = a0[:, D_IN:D_IN + 1] + a1[:, D_IN:D_IN + 1]
    neigh = sums / jnp.maximum(cnt, 1.0)
    r = lax.dot_general(h_ref[...], w_ref[:, :D_IN],
                        (((1,), (1,)), ((), ())),
                        preferred_element_type=jnp.float32)
    r = r + lax.dot_general(neigh, w_ref[:, D_IN:],
                            (((1,), (1,)), ((), ())),
                            preferred_element_type=jnp.float32)
    o_ref[...] = jnp.maximum(r + b_ref[...], 0.0)


def _tc_finish(h_pad, acc, W, b2, interpret=False):
    R = 1000
    grid = (NACC // R,)
    return pl.pallas_call(
        _tc_body,
        grid=grid,
        in_specs=[
            pl.BlockSpec((R, D_IN), lambda i: (i, 0)),
            pl.BlockSpec((NC, R, DEXT), lambda i: (0, i, 0)),
            pl.BlockSpec((D_IN, 2 * D_IN), lambda i: (0, 0)),
            pl.BlockSpec((1, D_OUT), lambda i: (0, 0)),
        ],
        out_specs=pl.BlockSpec((R, D_OUT), lambda i: (i, 0)),
        out_shape=jax.ShapeDtypeStruct((NACC, D_OUT), jnp.float32),
        interpret=interpret,
    )(h_pad, acc, W, b2)


def _hext_body(h_ref, o_ref):
    blk = h_ref[...].astype(jnp.bfloat16)
    ones = jnp.concatenate(
        [jnp.ones(blk.shape[:1] + (1,), jnp.bfloat16),
         jnp.zeros(blk.shape[:1] + (DEXT - D_IN - 1,), jnp.bfloat16)], axis=1)
    o_ref[...] = jnp.concatenate([blk, ones], axis=1)


def _hext_build(h):
    R = 1000
    return pl.pallas_call(
        _hext_body,
        grid=(N_NODES // R,),
        in_specs=[pl.BlockSpec((R, D_IN), lambda i: (i, 0))],
        out_specs=pl.BlockSpec((R, DEXT), lambda i: (i, 0)),
        out_shape=jax.ShapeDtypeStruct((N_NODES, DEXT), jnp.bfloat16),
    )(h)


def kernel(h, edge_index, W, b):
    idx4 = edge_index.astype(jnp.int32).reshape(2, NW, CPT, CH)
    hext = _hext_build(h)
    zeros_slab = jnp.zeros((ROWS_PER_TILE, DEXT), jnp.bfloat16)

    acc = _sc_aggregate(hext, idx4, zeros_slab)
    return _tc_finish(h, acc, W, b.reshape(1, D_OUT))


# speedup vs baseline: 3.6532x; 1.0288x over previous
import functools

import jax
import jax.numpy as jnp
from jax import lax
from jax.experimental import pallas as pl
from jax.experimental.pallas import tpu as pltpu
from jax.experimental.pallas import tpu_sc as plsc

N_NODES = 10000
N_EDGES = 320000
D_IN = 128
D_OUT = 128

NC = 2
NS = 16
NW = NC * NS
CH = 125
DEXT = D_IN + 16
NACC = N_NODES
ROWS_PER_TILE = NACC // NS
EPT = N_EDGES // NW
CPT = EPT // CH
NRB = 4


def _sc_body(hext_hbm, idx_hbm, zeros_hbm, feat_out, cnt_out,
             src_v, dst_v, rows, acc_sh, gsems, ssems):
    c = lax.axis_index("c")
    s = lax.axis_index("s")
    wid = s * NC + c
    my_rows = pl.ds(s * ROWS_PER_TILE, ROWS_PER_TILE)
    pltpu.sync_copy(zeros_hbm, acc_sh.at[my_rows])
    pltpu.sync_copy(idx_hbm.at[0, wid], src_v)
    pltpu.sync_copy(idx_hbm.at[1, wid], dst_v)
    plsc.subcore_barrier()

    def steps(j, dj):
        k = dj % NRB

        @pl.when((j >= 4) & (j < CPT + 4))
        def _drain_scatter():
            pltpu.make_async_copy(
                rows[k], acc_sh.at[dst_v.at[j - 4]], ssems[k]).wait()

        @pl.when(j < CPT)
        def _gather():
            pltpu.async_copy(hext_hbm.at[src_v.at[j]], rows[k], gsems[k])

        @pl.when((j >= 2) & (j < CPT + 2))
        def _scatter():
            kk = (dj - 2) % NRB
            pltpu.make_async_copy(
                hext_hbm.at[src_v.at[j - 2]], rows[kk], gsems[kk]).wait()
            pltpu.async_copy(rows[kk], acc_sh.at[dst_v.at[j - 2]],
                             ssems[kk], add=True)

    def block(bi, carry):
        j0 = bi * NRB
        for dj in range(NRB):
            steps(j0 + dj, dj)
        return carry

    lax.fori_loop(0, (CPT + 4 + NRB - 1) // NRB + 1, block, 0)
    plsc.subcore_barrier()
    pltpu.sync_copy(acc_sh.at[my_rows, pl.ds(0, D_IN)],
                    feat_out.at[c, my_rows])
    pltpu.sync_copy(acc_sh.at[my_rows, pl.ds(D_IN, DEXT - D_IN)],
                    cnt_out.at[c, my_rows])


def _sc_aggregate(hext, idx4, zeros_slab):
    mesh = plsc.VectorSubcoreMesh(core_axis_name="c", subcore_axis_name="s")
    f = pl.kernel(
        _sc_body,
        out_type=(jax.ShapeDtypeStruct((NC, NACC, D_IN), jnp.bfloat16),
                  jax.ShapeDtypeStruct((NC, NACC, DEXT - D_IN), jnp.bfloat16)),
        mesh=mesh,
        compiler_params=pltpu.CompilerParams(use_tc_tiling_on_sc=False),
        scratch_types=[
            pltpu.VMEM((CPT, CH), jnp.int32),
            pltpu.VMEM((CPT, CH), jnp.int32),
            [pltpu.VMEM((CH, DEXT), jnp.bfloat16) for _ in range(NRB)],
            pltpu.VMEM_SHARED((NACC, DEXT), jnp.bfloat16),
            [pltpu.SemaphoreType.DMA for _ in range(NRB)],
            [pltpu.SemaphoreType.DMA for _ in range(NRB)],
        ],
    )
    return f(hext, idx4, zeros_slab)


def _tc_body(h_ref, feat_ref, cnt_ref, w_ref, b_ref, o_ref):
    sums = feat_ref[0].astype(jnp.float32) + feat_ref[1].astype(jnp.float32)
    c0 = cnt_ref[0].astype(jnp.float32)
    c1 = cnt_ref[1].astype(jnp.float32)
    cnt = c0[:, 0:1] + c1[:, 0:1]
    neigh = sums / jnp.maximum(cnt, 1.0)
    r = lax.dot_general(h_ref[...], w_ref[:, :D_IN],
                        (((1,), (1,)), ((), ())),
                        preferred_element_type=jnp.float32)
    r = r + lax.dot_general(neigh, w_ref[:, D_IN:],
                            (((1,), (1,)), ((), ())),
                            preferred_element_type=jnp.float32)
    o_ref[...] = jnp.maximum(r + b_ref[...], 0.0)


def _tc_finish(h_pad, feat, cnt, W, b2, interpret=False):
    R = 1000
    grid = (NACC // R,)
    return pl.pallas_call(
        _tc_body,
        grid=grid,
        in_specs=[
            pl.BlockSpec((R, D_IN), lambda i: (i, 0)),
            pl.BlockSpec((NC, R, D_IN), lambda i: (0, i, 0)),
            pl.BlockSpec((NC, R, DEXT - D_IN), lambda i: (0, i, 0)),
            pl.BlockSpec((D_IN, 2 * D_IN), lambda i: (0, 0)),
            pl.BlockSpec((1, D_OUT), lambda i: (0, 0)),
        ],
        out_specs=pl.BlockSpec((R, D_OUT), lambda i: (i, 0)),
        out_shape=jax.ShapeDtypeStruct((NACC, D_OUT), jnp.float32),
        interpret=interpret,
    )(h_pad, feat, cnt, W, b2)


def _hext_body(h_ref, o_ref):
    blk = h_ref[...].astype(jnp.bfloat16)
    ones = jnp.concatenate(
        [jnp.ones(blk.shape[:1] + (1,), jnp.bfloat16),
         jnp.zeros(blk.shape[:1] + (DEXT - D_IN - 1,), jnp.bfloat16)], axis=1)
    o_ref[...] = jnp.concatenate([blk, ones], axis=1)


def _hext_build(h):
    R = 1000
    return pl.pallas_call(
        _hext_body,
        grid=(N_NODES // R,),
        in_specs=[pl.BlockSpec((R, D_IN), lambda i: (i, 0))],
        out_specs=pl.BlockSpec((R, DEXT), lambda i: (i, 0)),
        out_shape=jax.ShapeDtypeStruct((N_NODES, DEXT), jnp.bfloat16),
    )(h)


def kernel(h, edge_index, W, b):
    idx4 = edge_index.astype(jnp.int32).reshape(2, NW, CPT, CH)
    hext = _hext_build(h)
    zeros_slab = jnp.zeros((ROWS_PER_TILE, DEXT), jnp.bfloat16)

    feat, cnt = _sc_aggregate(hext, idx4, zeros_slab)
    return _tc_finish(h, feat, cnt, W, b.reshape(1, D_OUT))
